# baseline (speedup 1.0000x reference)
import jax
import jax.numpy as jnp
from jax import lax
from jax.experimental import pallas as pl
from jax.experimental.pallas import tpu as pltpu
from jax.experimental.pallas import tpu_sc as plsc

N_NODES = 10000
N_PAD = 10240
PAD_NODE = N_PAD - 1
N_EDGES = 320000
N_WORKERS = 32
E_PER_W = 10240
CHUNK = 256
N_CHUNKS = E_PER_W // CHUNK
RING = 2
WIDE = 128
HALF = N_PAD // 2
ACC_R = 5376
DUMP = ACC_R - 1
ROWS_T = ACC_R // 16
ROWS_H = ROWS_T // 2
N_P2 = HALF + ACC_R

_MESH = plsc.VectorSubcoreMesh(core_axis_name="c", subcore_axis_name="s")


def _deg_body(dst_hbm, zero_hbm, out_hbm, dstv, table):
    c = lax.axis_index("c")
    s = lax.axis_index("s")
    wid = c * 16 + s
    pltpu.sync_copy(zero_hbm, table)
    pltpu.sync_copy(dst_hbm.at[wid], dstv)
    ones = jnp.zeros((16,), jnp.float32) + 1.0

    def step(j, carry):
        iv = dstv[pl.ds(j * 16, 16)]
        plsc.addupdate_scatter(table, [iv], ones)
        return carry

    lax.fori_loop(0, E_PER_W // 16, step, 0)
    pltpu.sync_copy(table, out_hbm.at[wid])


_deg_kernel = pl.kernel(
    _deg_body,
    out_type=jax.ShapeDtypeStruct((N_WORKERS, N_PAD), jnp.float32),
    mesh=_MESH,
    scratch_types=[
        pltpu.VMEM((E_PER_W,), jnp.int32),
        pltpu.VMEM((N_PAD,), jnp.float32),
    ],
    compiler_params=pltpu.CompilerParams(needs_layout_passes=False),
)


def _agg_body(base, src_hbm, dst_hbm, p_hbm, out_hbm, srcv0, srcv1,
              dstv, rows0, rows1, stage, acc, gsem):
    c = lax.axis_index("c")
    s = lax.axis_index("s")
    wid = c * 16 + s
    srcv = [srcv0, srcv1]
    rows = [rows0, rows1]
    for h in range(2):
        r0 = s * ROWS_T + h * ROWS_H
        pltpu.sync_copy(p_hbm.at[pl.ds(base + r0, ROWS_H)], stage)
        pltpu.sync_copy(stage, acc.at[pl.ds(r0, ROWS_H)])
    plsc.subcore_barrier()

    def g_start(q, b):
        pltpu.sync_copy(src_hbm.at[wid, q], srcv[b])
        pltpu.async_copy(p_hbm.at[srcv[b]], rows[b], gsem)

    def g_wait(q, b):
        pltpu.make_async_copy(p_hbm.at[srcv[b]], rows[b], gsem).wait()

    for b in range(RING):
        g_start(b, b)
    for q in range(N_CHUNKS):
        b = q % RING
        g_wait(q, b)
        pltpu.sync_copy(dst_hbm.at[wid, q], dstv)
        pltpu.sync_copy(rows[b], acc.at[dstv], add=True)
        if q + RING < N_CHUNKS:
            g_start(q + RING, b)
    plsc.subcore_barrier()
    for h in range(2):
        r0 = s * ROWS_T + h * ROWS_H
        pltpu.sync_copy(acc.at[pl.ds(r0, ROWS_H)], stage)
        pltpu.sync_copy(stage, out_hbm.at[c, pl.ds(r0, ROWS_H)])


def _make_agg(base):
    import functools
    return pl.kernel(
        functools.partial(_agg_body, base),
        out_type=jax.ShapeDtypeStruct((2, ACC_R, WIDE), jnp.float32),
        mesh=_MESH,
        scratch_types=[
            pltpu.VMEM((CHUNK,), jnp.int32),
            pltpu.VMEM((CHUNK,), jnp.int32),
            pltpu.VMEM((CHUNK,), jnp.int32),
            pltpu.VMEM((CHUNK, WIDE), jnp.float32),
            pltpu.VMEM((CHUNK, WIDE), jnp.float32),
            pltpu.VMEM((ROWS_H, WIDE), jnp.float32),
            pltpu.VMEM_SHARED((ACC_R, WIDE), jnp.float32),
            pltpu.SemaphoreType.DMA,
        ],
    )


_agg_lo = _make_agg(0)
_agg_hi = _make_agg(HALF)


def _tc1_body(deg2_ref, x_ref, w1_ref, p1_ref, dinv_ref):
    deg = jnp.sum(deg2_ref[...], axis=0, keepdims=True) + 1.0
    dinv = jnp.transpose(lax.rsqrt(deg), (1, 0))
    h = jnp.dot(x_ref[...], w1_ref[...], preferred_element_type=jnp.float32)
    p1_ref[...] = jnp.zeros((N_P2, WIDE), jnp.float32)
    p1_ref[:N_NODES, :20] = h * dinv[:N_NODES, :]
    dinv_ref[...] = dinv


def _tc2_body(accA_ref, accB_ref, p1_ref, dinv_ref, b1_ref, wmu_ref, p2_ref):
    qlo = accA_ref[0] + accA_ref[1] - p1_ref[:ACC_R]
    qhi = accB_ref[0] + accB_ref[1] - p1_ref[HALF:HALF + ACC_R]
    q1 = jnp.concatenate([qlo[:HALF, :20], qhi[:HALF, :20]], axis=0)
    h1 = jnp.maximum(q1 * dinv_ref[...] + b1_ref[...], 0.0)
    h2 = jnp.dot(h1, wmu_ref[...], preferred_element_type=jnp.float32)
    p2_ref[...] = jnp.zeros((N_P2, WIDE), jnp.float32)
    p2_ref[:N_PAD, :10] = h2 * dinv_ref[...]


def _tc3_body(accA_ref, accB_ref, p2_ref, dinv_ref, bmu_ref, z_ref):
    zlo = accA_ref[0] + accA_ref[1] - p2_ref[:ACC_R]
    zhi = accB_ref[0] + accB_ref[1] - p2_ref[HALF:HALF + ACC_R]
    z = jnp.concatenate([zlo[:HALF, :10], zhi[:HALF, :10]], axis=0)
    z = z * dinv_ref[...] + bmu_ref[...]
    z_ref[...] = z[:N_NODES, :]


def _shard_edges(e):
    real = e.reshape(N_WORKERS, N_EDGES // N_WORKERS)
    pad = jnp.full((N_WORKERS, E_PER_W - N_EDGES // N_WORKERS), PAD_NODE,
                   jnp.int32)
    return jnp.concatenate([real, pad], axis=1).reshape(
        N_WORKERS, N_CHUNKS, CHUNK)


@jax.jit
def kernel(x, edge_index, W1, b1, W_mu, b_mu, W_ls, b_ls):
    del W_ls, b_ls
    ei = edge_index.astype(jnp.int32)
    src_r = _shard_edges(ei[0])
    dst_r = _shard_edges(ei[1])
    dstA = jnp.where(dst_r < HALF, dst_r, DUMP)
    dstB = jnp.where(dst_r >= HALF, dst_r - HALF, DUMP)
    dst_flat = dst_r.reshape(N_WORKERS, E_PER_W)
    zero_n = jnp.zeros((N_PAD,), jnp.float32)

    deg2 = _deg_kernel(dst_flat, zero_n)

    p1, dinv = pl.pallas_call(
        _tc1_body,
        out_shape=(jax.ShapeDtypeStruct((N_P2, WIDE), jnp.float32),
                   jax.ShapeDtypeStruct((N_PAD, 1), jnp.float32)),
    )(deg2, x, W1)

    acc1A = _agg_lo(src_r, dstA, p1)
    acc1B = _agg_hi(src_r, dstB, p1)

    p2 = pl.pallas_call(
        _tc2_body,
        out_shape=jax.ShapeDtypeStruct((N_P2, WIDE), jnp.float32),
    )(acc1A, acc1B, p1, dinv, b1.reshape(1, 20), W_mu)

    acc2A = _agg_lo(src_r, dstA, p2)
    acc2B = _agg_hi(src_r, dstB, p2)

    z = pl.pallas_call(
        _tc3_body,
        out_shape=jax.ShapeDtypeStruct((N_NODES, 10), jnp.float32),
    )(acc2A, acc2B, p2, dinv, b_mu.reshape(1, 10))

    return z


# speedup vs baseline: 1.0001x; 1.0001x over previous
import functools

import jax
import jax.numpy as jnp
from jax import lax
from jax.experimental import pallas as pl
from jax.experimental.pallas import tpu as pltpu
from jax.experimental.pallas import tpu_sc as plsc

N_NODES = 10000
N_PAD = 10240
PAD_NODE = N_PAD - 1
N_EDGES = 320000
N_WORKERS = 32
E_PER_W = 10240
CHUNK = 256
N_CHUNKS = E_PER_W // CHUNK
RING = 2
WIDE = 128
HALF = N_PAD // 2
ACC_R = 5376
DUMP = ACC_R - 1
ROWS_T = ACC_R // 16
ROWS_H = ROWS_T // 2
N_P2 = HALF + ACC_R

_MESH = plsc.VectorSubcoreMesh(core_axis_name="c", subcore_axis_name="s")


def _deg_body(dst_hbm, zero_hbm, out_hbm, dstv, table):
    c = lax.axis_index("c")
    s = lax.axis_index("s")
    wid = c * 16 + s
    pltpu.sync_copy(zero_hbm, table)
    pltpu.sync_copy(dst_hbm.at[wid], dstv)
    ones = jnp.zeros((16,), jnp.float32) + 1.0

    def step(j, carry):
        iv = dstv[pl.ds(j * 16, 16)]
        plsc.addupdate_scatter(table, [iv], ones)
        return carry

    lax.fori_loop(0, E_PER_W // 16, step, 0)
    pltpu.sync_copy(table, out_hbm.at[wid])


_deg_kernel = pl.kernel(
    _deg_body,
    out_type=jax.ShapeDtypeStruct((N_WORKERS, N_PAD), jnp.float32),
    mesh=_MESH,
    scratch_types=[
        pltpu.VMEM((E_PER_W,), jnp.int32),
        pltpu.VMEM((N_PAD,), jnp.float32),
    ],
    compiler_params=pltpu.CompilerParams(needs_layout_passes=False),
)


def _agg_body(base, src_hbm, dst_hbm, p_hbm, out_hbm, srcv0, srcv1,
              dstv, rows0, rows1, stage, acc, gsem):
    c = lax.axis_index("c")
    s = lax.axis_index("s")
    wid = c * 16 + s
    srcv = [srcv0, srcv1]
    rows = [rows0, rows1]
    for h in range(2):
        r0 = s * ROWS_T + h * ROWS_H
        pltpu.sync_copy(p_hbm.at[pl.ds(base + r0, ROWS_H)], stage)
        pltpu.sync_copy(stage, acc.at[pl.ds(r0, ROWS_H)])
    plsc.subcore_barrier()

    def g_start(q, b):
        pltpu.sync_copy(src_hbm.at[wid, q], srcv[b])
        pltpu.async_copy(p_hbm.at[srcv[b]], rows[b], gsem)

    def g_wait(q, b):
        pltpu.make_async_copy(p_hbm.at[srcv[b]], rows[b], gsem).wait()

    for b in range(RING):
        g_start(b, b)
    for q in range(N_CHUNKS):
        b = q % RING
        g_wait(q, b)
        pltpu.sync_copy(dst_hbm.at[wid, q], dstv)
        pltpu.sync_copy(rows[b], acc.at[dstv], add=True)
        if q + RING < N_CHUNKS:
            g_start(q + RING, b)
    plsc.subcore_barrier()
    for h in range(2):
        r0 = s * ROWS_T + h * ROWS_H
        pltpu.sync_copy(acc.at[pl.ds(r0, ROWS_H)], stage)
        pltpu.sync_copy(stage, out_hbm.at[c, pl.ds(r0, ROWS_H)])


def _make_agg(base):
    return pl.kernel(
        functools.partial(_agg_body, base),
        out_type=jax.ShapeDtypeStruct((2, ACC_R, WIDE), jnp.float32),
        mesh=_MESH,
        scratch_types=[
            pltpu.VMEM((CHUNK,), jnp.int32),
            pltpu.VMEM((CHUNK,), jnp.int32),
            pltpu.VMEM((CHUNK,), jnp.int32),
            pltpu.VMEM((CHUNK, WIDE), jnp.float32),
            pltpu.VMEM((CHUNK, WIDE), jnp.float32),
            pltpu.VMEM((ROWS_H, WIDE), jnp.float32),
            pltpu.VMEM_SHARED((ACC_R, WIDE), jnp.float32),
            pltpu.SemaphoreType.DMA,
        ],
    )


_agg_lo = _make_agg(0)
_agg_hi = _make_agg(HALF)


def _tc1_body(deg2_ref, x_ref, w1_ref, p1_ref, dinv_ref):
    deg = jnp.sum(deg2_ref[...], axis=0, keepdims=True) + 1.0
    dinv = jnp.transpose(lax.rsqrt(deg), (1, 0))
    h = jnp.dot(x_ref[...], w1_ref[...], preferred_element_type=jnp.float32)
    p1_ref[...] = jnp.zeros((N_P2, WIDE), jnp.float32)
    p1_ref[:N_NODES, :20] = h * dinv[:N_NODES, :]
    dinv_ref[...] = dinv


def _tc2_body(accA_ref, accB_ref, p1_ref, dinv_ref, b1_ref, wmu_ref, p2_ref):
    qlo = accA_ref[0] + accA_ref[1] - p1_ref[:ACC_R]
    qhi = accB_ref[0] + accB_ref[1] - p1_ref[HALF:HALF + ACC_R]
    q1 = jnp.concatenate([qlo[:HALF, :20], qhi[:HALF, :20]], axis=0)
    h1 = jnp.maximum(q1 * dinv_ref[...] + b1_ref[...], 0.0)
    h2 = jnp.dot(h1, wmu_ref[...], preferred_element_type=jnp.float32)
    p2_ref[...] = jnp.zeros((N_P2, WIDE), jnp.float32)
    p2_ref[:N_PAD, :10] = h2 * dinv_ref[...]


def _tc3_body(accA_ref, accB_ref, p2_ref, dinv_ref, bmu_ref, z_ref):
    zlo = accA_ref[0] + accA_ref[1] - p2_ref[:ACC_R]
    zhi = accB_ref[0] + accB_ref[1] - p2_ref[HALF:HALF + ACC_R]
    z = jnp.concatenate([zlo[:HALF, :10], zhi[:HALF, :10]], axis=0)
    z = z * dinv_ref[...] + bmu_ref[...]
    z_ref[...] = z[:N_NODES, :]


def _shard_edges(e):
    real = e.reshape(N_WORKERS, N_EDGES // N_WORKERS)
    pad = jnp.full((N_WORKERS, E_PER_W - N_EDGES // N_WORKERS), PAD_NODE,
                   jnp.int32)
    return jnp.concatenate([real, pad], axis=1).reshape(
        N_WORKERS, N_CHUNKS, CHUNK)


@jax.jit
def kernel(x, edge_index, W1, b1, W_mu, b_mu, W_ls, b_ls):
    del W_ls, b_ls
    ei = edge_index.astype(jnp.int32)
    src_r = _shard_edges(ei[0])
    dst_r = _shard_edges(ei[1])
    dstA = jnp.where(dst_r < HALF, dst_r, DUMP)
    dstB = jnp.where(dst_r >= HALF, dst_r - HALF, DUMP)
    dst_flat = dst_r.reshape(N_WORKERS, E_PER_W)
    zero_n = jnp.zeros((N_PAD,), jnp.float32)

    deg2 = _deg_kernel(dst_flat, zero_n)

    p1, dinv = pl.pallas_call(
        _tc1_body,
        out_shape=(jax.ShapeDtypeStruct((N_P2, WIDE), jnp.float32),
                   jax.ShapeDtypeStruct((N_PAD, 1), jnp.float32)),
    )(deg2, x, W1)

    acc1A = _agg_lo(src_r, dstA, p1)
    acc1B = _agg_hi(src_r, dstB, p1)

    p2 = pl.pallas_call(
        _tc2_body,
        out_shape=jax.ShapeDtypeStruct((N_P2, WIDE), jnp.float32),
    )(acc1A, acc1B, p1, dinv, b1.reshape(1, 20), W_mu)

    acc2A = _agg_lo(src_r, dstA, p2)
    acc2B = _agg_hi(src_r, dstB, p2)

    z = pl.pallas_call(
        _tc3_body,
        out_shape=jax.ShapeDtypeStruct((N_NODES, 10), jnp.float32),
    )(acc2A, acc2B, p2, dinv, b_mu.reshape(1, 10))

    return z
